# baseline (speedup 1.0000x reference)
import functools

import jax
import jax.numpy as jnp
from jax import lax
from jax.experimental import pallas as pl
from jax.experimental.pallas import tpu as pltpu
from jax.experimental.pallas import tpu_sc as plsc

B = 16384
D = 64
L = 16
NW = 32
BPW = B // NW
CHUNK = 128
NCH = BPW // CHUNK


def _mf_body(uids_hbm, iids_hbm, uemb_hbm, iemb_hbm, ubias_hbm, ibias_hbm,
             out_hbm, uidx_v, iidx_v, urows_v, irows_v, ub_v, ib_v, out_v,
             sem):
    wid = lax.axis_index("s") * 2 + lax.axis_index("c")
    base = wid * BPW

    pltpu.sync_copy(uids_hbm.at[pl.ds(base, BPW)], uidx_v)
    pltpu.sync_copy(iids_hbm.at[pl.ds(base, BPW)], iidx_v)

    copies = []
    for c in range(NCH):
        s = c * CHUNK
        sl = pl.ds(s, CHUNK)
        copies.append(pltpu.make_async_copy(
            uemb_hbm.at[uidx_v.at[sl]], urows_v.at[sl], sem))
        copies.append(pltpu.make_async_copy(
            iemb_hbm.at[iidx_v.at[sl]], irows_v.at[sl], sem))
        copies.append(pltpu.make_async_copy(
            ubias_hbm.at[uidx_v.at[sl]], ub_v.at[sl], sem))
        copies.append(pltpu.make_async_copy(
            ibias_hbm.at[iidx_v.at[sl]], ib_v.at[sl], sem))
    for cp in copies:
        cp.start()
    for cp in copies:
        cp.wait()

    lane = lax.broadcasted_iota(jnp.int32, (L,), 0)
    perms = [lane ^ (1 << st) for st in range(4)]

    def group(g, carry):
        gb = g * L
        res = jnp.zeros((L,), jnp.float32)
        for r in range(L):
            row = gb + r
            acc = urows_v[row, pl.ds(0, L)] * irows_v[row, pl.ds(0, L)]
            for k in range(1, D // L):
                acc = acc + (urows_v[row, pl.ds(k * L, L)] *
                             irows_v[row, pl.ds(k * L, L)])
            for p in perms:
                acc = acc + acc.at[p].get(mode="promise_in_bounds")
            res = jnp.where(lane == r, acc, res)
        sl16 = pl.ds(gb, L)
        out_v[sl16] = res + ub_v[sl16] + ib_v[sl16]
        return carry

    lax.fori_loop(0, BPW // L, group, 0)
    pltpu.sync_copy(out_v, out_hbm.at[pl.ds(base, BPW)])


@functools.partial(
    pl.kernel,
    mesh=plsc.VectorSubcoreMesh(core_axis_name="c", subcore_axis_name="s"),
    out_type=jax.ShapeDtypeStruct((B,), jnp.float32),
    compiler_params=pltpu.CompilerParams(use_tc_tiling_on_sc=False),
    scratch_types=[
        pltpu.VMEM((BPW,), jnp.int32),
        pltpu.VMEM((BPW,), jnp.int32),
        pltpu.VMEM((BPW, D), jnp.float32),
        pltpu.VMEM((BPW, D), jnp.float32),
        pltpu.VMEM((BPW,), jnp.float32),
        pltpu.VMEM((BPW,), jnp.float32),
        pltpu.VMEM((BPW,), jnp.float32),
        pltpu.SemaphoreType.DMA,
    ],
)
def _mf_kernel(*refs):
    _mf_body(*refs)


def kernel(user_ids, item_ids, user_emb, item_emb, user_bias, item_bias):
    return _mf_kernel(user_ids.astype(jnp.int32), item_ids.astype(jnp.int32),
                      user_emb, item_emb,
                      user_bias.reshape(-1), item_bias.reshape(-1))


# speedup vs baseline: 1.3999x; 1.3999x over previous
import functools

import jax
import jax.numpy as jnp
from jax import lax
from jax.experimental import pallas as pl
from jax.experimental.pallas import tpu as pltpu
from jax.experimental.pallas import tpu_sc as plsc

B = 16384
D = 64
L = 16
NW = 32
BPW = B // NW
CH = 128
NCH = BPW // CH


def _mf_body(uids_hbm, iids_hbm, uemb_hbm, iemb_hbm, ubias_hbm, ibias_hbm,
             out_hbm, uidx_v, iidx_v,
             u0, u1, i0, i1, ub_v, ib_v, out_v, sem0, sem1, bsem):
    wid = lax.axis_index("s") * 2 + lax.axis_index("c")
    base = wid * BPW

    pltpu.sync_copy(uids_hbm.at[pl.ds(base, BPW)], uidx_v)
    pltpu.sync_copy(iids_hbm.at[pl.ds(base, BPW)], iidx_v)

    bias_copies = []
    for c in range(NCH):
        sl = pl.ds(c * CH, CH)
        bias_copies.append(pltpu.make_async_copy(
            ubias_hbm.at[uidx_v.at[sl]], ub_v.at[sl], bsem))
        bias_copies.append(pltpu.make_async_copy(
            ibias_hbm.at[iidx_v.at[sl]], ib_v.at[sl], bsem))
    for cp in bias_copies:
        cp.start()

    ubufs, ibufs, sems = (u0, u1), (i0, i1), (sem0, sem1)

    def fire(c, buf):
        ub, ib, sm = ubufs[buf], ibufs[buf], sems[buf]

        def fetch(g, carry):
            uvec = uidx_v[pl.ds(c * CH + g * L, L)]
            ivec = iidx_v[pl.ds(c * CH + g * L, L)]
            for r in range(L):
                pltpu.make_async_copy(uemb_hbm.at[uvec[r]],
                                      ub.at[g * L + r], sm).start()
                pltpu.make_async_copy(iemb_hbm.at[ivec[r]],
                                      ib.at[g * L + r], sm).start()
            return carry

        lax.fori_loop(0, CH // L, fetch, 0)

    def drain(buf):
        sm = sems[buf]
        pltpu.make_async_copy(uemb_hbm.at[pl.ds(0, CH)], ubufs[buf], sm).wait()
        pltpu.make_async_copy(iemb_hbm.at[pl.ds(0, CH)], ibufs[buf], sm).wait()

    lane = lax.broadcasted_iota(jnp.int32, (L,), 0)
    perms = [lane ^ (1 << st) for st in range(4)]

    def compute(c, buf):
        urows, irows = ubufs[buf], ibufs[buf]

        def group(g, carry):
            gb = g * L
            res = jnp.zeros((L,), jnp.float32)
            for r in range(L):
                row = gb + r
                acc = urows[row, pl.ds(0, L)] * irows[row, pl.ds(0, L)]
                for k in range(1, D // L):
                    acc = acc + (urows[row, pl.ds(k * L, L)] *
                                 irows[row, pl.ds(k * L, L)])
                for p in perms:
                    acc = acc + acc.at[p].get(mode="promise_in_bounds")
                res = jnp.where(lane == r, acc, res)
            out_v[pl.ds(c * CH + gb, L)] = res
            return carry

        lax.fori_loop(0, CH // L, group, 0)

    fire(0, 0)
    for c in range(NCH):
        if c + 1 < NCH:
            fire(c + 1, (c + 1) % 2)
        drain(c % 2)
        compute(c, c % 2)

    for cp in bias_copies:
        cp.wait()

    def biased(g, carry):
        sl16 = pl.ds(g * L, L)
        out_v[sl16] = out_v[sl16] + ub_v[sl16] + ib_v[sl16]
        return carry

    lax.fori_loop(0, BPW // L, biased, 0)

    pltpu.sync_copy(out_v, out_hbm.at[pl.ds(base, BPW)])


@functools.partial(
    pl.kernel,
    mesh=plsc.VectorSubcoreMesh(core_axis_name="c", subcore_axis_name="s"),
    out_type=jax.ShapeDtypeStruct((B,), jnp.float32),
    compiler_params=pltpu.CompilerParams(use_tc_tiling_on_sc=True),
    scratch_types=[
        pltpu.VMEM((BPW,), jnp.int32),
        pltpu.VMEM((BPW,), jnp.int32),
        pltpu.VMEM((CH, D), jnp.float32),
        pltpu.VMEM((CH, D), jnp.float32),
        pltpu.VMEM((CH, D), jnp.float32),
        pltpu.VMEM((CH, D), jnp.float32),
        pltpu.VMEM((BPW,), jnp.float32),
        pltpu.VMEM((BPW,), jnp.float32),
        pltpu.VMEM((BPW,), jnp.float32),
        pltpu.SemaphoreType.DMA,
        pltpu.SemaphoreType.DMA,
        pltpu.SemaphoreType.DMA,
    ],
)
def _mf_kernel(*refs):
    _mf_body(*refs)


def kernel(user_ids, item_ids, user_emb, item_emb, user_bias, item_bias):
    return _mf_kernel(user_ids.astype(jnp.int32), item_ids.astype(jnp.int32),
                      user_emb, item_emb,
                      user_bias.reshape(-1), item_bias.reshape(-1))
